# baseline (speedup 1.0000x reference)
import functools

import jax
import jax.numpy as jnp
from jax import lax
from jax.experimental import pallas as pl
from jax.experimental.pallas import tpu as pltpu
from jax.experimental.pallas import tpu_sc as plsc

N = 10000
E = 320000
D = 128
H = 32
WPAD = 48
NC, NS, L = 2, 16, 16
NW = NC * NS
EPW = E // NW
SB = 80
K = 5
B = SB * K
NB = EPW // B
NPAD = 10240
RPS = NPAD // NS

R = 400
GRID = N // R


def _pre_kernel(xc_ref, xn_ref, wc_ref, bc_ref, wn_ref, bn_ref, wa_ref,
                ba_ref, nbrp_ref, acur_ref, anbr_ref):
    wa = wa_ref[...]
    wa1 = wa[:H, :]
    wa2 = wa[H:, :]
    cur = jnp.dot(xc_ref[...], wc_ref[...],
                  preferred_element_type=jnp.float32) + bc_ref[...][None, :]
    nbr = jnp.dot(xn_ref[...], wn_ref[...],
                  preferred_element_type=jnp.float32) + bn_ref[...][None, :]
    acur = jnp.dot(cur, wa1, preferred_element_type=jnp.float32) + ba_ref[...]
    anbr = jnp.dot(nbr, wa2, preferred_element_type=jnp.float32)
    nbrp_ref[...] = nbr
    acur_ref[...] = acur
    anbr_ref[...] = anbr


def _pre(x_cur, x_nbr, W_cur, b_cur, W_nbr, b_nbr, W_att, b_att):
    return pl.pallas_call(
        _pre_kernel,
        out_shape=[
            jax.ShapeDtypeStruct((N, H), jnp.float32),
            jax.ShapeDtypeStruct((N, 1), jnp.float32),
            jax.ShapeDtypeStruct((N, 1), jnp.float32),
        ],
    )(x_cur, x_nbr, W_cur, b_cur, W_nbr, b_nbr, W_att, b_att)


def _sc_edge_kernel(acur_hbm, anbr_hbm, nbrp_hbm, src_hbm, dst_hbm, out_hbm,
                    acur_v, srcb_v, dstb_v, rows0, rows1, abuf0, abuf1,
                    sbuf, wsum, g0, g1, s0, s1):
    cid = lax.axis_index("c")
    sid = lax.axis_index("s")
    wid = cid * NS + sid
    stage = [pltpu.async_copy(acur_hbm, acur_v, g0),
             pltpu.async_copy(src_hbm.at[wid], srcb_v, g0),
             pltpu.async_copy(dst_hbm.at[wid], dstb_v, g0)]
    zero16 = jnp.zeros((L,), jnp.float32)

    def zrow(r, c):
        for cc in range(WPAD // L):
            sbuf[r, pl.ds(cc * L, L)] = zero16
        return c

    lax.fori_loop(0, B, zrow, 0)
    zc = [pltpu.async_copy(sbuf.at[pl.ds(0, SB)],
                           wsum.at[pl.ds(sid * RPS + k * SB, SB)], g1)
          for k in range(RPS // SB)]
    for d in stage + zc:
        d.wait()
    plsc.subcore_barrier()

    def fire_gathers(b, buf, abuf, sem):
        for k in range(K):
            pltpu.async_copy(nbrp_hbm.at[dstb_v.at[b, k]],
                             buf.at[pl.ds(k * SB, SB)], sem)
            pltpu.async_copy(anbr_hbm.at[dstb_v.at[b, k]],
                             abuf.at[pl.ds(k * SB, SB)], sem)

    def drain_gathers(b, buf, abuf, sem):
        for k in range(K):
            pltpu.make_async_copy(nbrp_hbm.at[dstb_v.at[b, k]],
                                  buf.at[pl.ds(k * SB, SB)], sem).wait()
            pltpu.make_async_copy(anbr_hbm.at[dstb_v.at[b, k]],
                                  abuf.at[pl.ds(k * SB, SB)], sem).wait()

    def fire_scatters(b, buf, sem):
        for k in range(K):
            pltpu.async_copy(buf.at[pl.ds(k * SB, SB)],
                             wsum.at[srcb_v.at[b, k]], sem, add=True)

    def drain_scatters(b, buf, sem):
        for k in range(K):
            pltpu.make_async_copy(buf.at[pl.ds(k * SB, SB)],
                                  wsum.at[srcb_v.at[b, k]], sem).wait()

    iota16 = lax.iota(jnp.int32, L)
    c32 = jnp.full((L,), H, jnp.int32)
    c33 = jnp.full((L,), H + 1, jnp.int32)

    def compute(b, buf, abuf):
        def grp(g):
            evec = iota16 + g * L
            src16 = srcb_v[b, g // (SB // L), pl.ds((g % (SB // L)) * L, L)]
            sv = plsc.load_gather(acur_v, [src16])
            dv = abuf[pl.ds(g * L, L)]
            s = sv + dv
            s = jnp.where(s >= 0.0, s, 0.3 * s)
            wv = jnp.exp(s)
            plsc.store_scatter(sbuf, [evec, c32], wv)

            def colbody(j):
                cj = jnp.full((L,), 0, jnp.int32) + j
                col = plsc.load_gather(buf, [evec, cj])
                plsc.store_scatter(sbuf, [evec, cj], col * wv)

            plsc.parallel_loop(0, H, unroll=8)(colbody)

        plsc.parallel_loop(0, B // L, unroll=5)(grp)

    fire_gathers(0, rows0, abuf0, g0)
    fire_gathers(1, rows1, abuf1, g1)
    drain_gathers(0, rows0, abuf0, g0)
    compute(0, rows0, abuf0)
    fire_scatters(0, sbuf, s0)

    def pipe(p, c):
        b0 = 2 * p
        fire_gathers(b0 + 2, rows0, abuf0, g0)
        drain_gathers(b0 + 1, rows1, abuf1, g1)
        drain_scatters(b0, sbuf, s0)
        compute(b0 + 1, rows1, abuf1)
        fire_scatters(b0 + 1, sbuf, s0)
        fire_gathers(b0 + 3, rows1, abuf1, g1)
        drain_gathers(b0 + 2, rows0, abuf0, g0)
        drain_scatters(b0 + 1, sbuf, s0)
        compute(b0 + 2, rows0, abuf0)
        fire_scatters(b0 + 2, sbuf, s0)
        return c

    lax.fori_loop(0, (NB - 3) // 2, pipe, 0)
    fire_gathers(NB - 1, rows0, abuf0, g0)
    drain_gathers(NB - 2, rows1, abuf1, g1)
    drain_scatters(NB - 3, sbuf, s0)
    compute(NB - 2, rows1, abuf1)
    fire_scatters(NB - 2, sbuf, s0)
    drain_gathers(NB - 1, rows0, abuf0, g0)
    drain_scatters(NB - 2, sbuf, s0)
    compute(NB - 1, rows0, abuf0)
    fire_scatters(NB - 1, sbuf, s0)
    drain_scatters(NB - 1, sbuf, s0)

    plsc.subcore_barrier()
    pltpu.sync_copy(wsum.at[pl.ds(sid * RPS, RPS)],
                    out_hbm.at[cid, pl.ds(sid * RPS, RPS)])


_sc_edges = functools.partial(
    pl.kernel,
    out_type=jax.ShapeDtypeStruct((NC, NPAD, WPAD), jnp.float32),
    mesh=plsc.VectorSubcoreMesh(core_axis_name="c", subcore_axis_name="s"),
    compiler_params=pltpu.CompilerParams(needs_layout_passes=False,
                                         use_tc_tiling_on_sc=False),
    scratch_types=[
        pltpu.VMEM((N,), jnp.float32),
        pltpu.VMEM((NB, K, SB), jnp.int32),
        pltpu.VMEM((NB, K, SB), jnp.int32),
        pltpu.VMEM((B, H), jnp.float32),
        pltpu.VMEM((B, H), jnp.float32),
        pltpu.VMEM((B,), jnp.float32),
        pltpu.VMEM((B,), jnp.float32),
        pltpu.VMEM((B, WPAD), jnp.float32),
        pltpu.VMEM_SHARED((NPAD, WPAD), jnp.float32),
        pltpu.SemaphoreType.DMA,
        pltpu.SemaphoreType.DMA,
        pltpu.SemaphoreType.DMA,
        pltpu.SemaphoreType.DMA,
    ],
)(_sc_edge_kernel)


def _post_kernel(p_ref, o_ref):
    p = p_ref[0, :N] + p_ref[1, :N]
    den = p[:, H:H + 1]
    num = p[:, 0:H]
    o_ref[...] = jnp.where(den > 0.0, num / den, 0.0)


def _post(parts):
    return pl.pallas_call(
        _post_kernel,
        out_shape=jax.ShapeDtypeStruct((N, H), jnp.float32),
    )(parts)


def kernel(x_cur, x_nbr, W_cur, b_cur, W_nbr, b_nbr, W_att, b_att, edges):
    nbrp, acur2d, anbr2d = _pre(x_cur, x_nbr, W_cur, b_cur, W_nbr, b_nbr,
                                W_att, b_att)
    acur = acur2d.reshape(N)
    anbr = anbr2d.reshape(N)
    src = edges[:, 0].reshape(NW, NB, K, SB)
    dst = edges[:, 1].reshape(NW, NB, K, SB)
    parts = _sc_edges(acur, anbr, nbrp, src, dst)
    return _post(parts)


# speedup vs baseline: 1.5553x; 1.5553x over previous
import functools

import jax
import jax.numpy as jnp
from jax import lax
from jax.experimental import pallas as pl
from jax.experimental.pallas import tpu as pltpu
from jax.experimental.pallas import tpu_sc as plsc

N = 10000
E = 320000
D = 128
H = 32
WPAD = 48
NC, NS, L = 2, 16, 16
NW = NC * NS
EPW = E // NW
SB = 80
K = 5
B = SB * K
NB = EPW // B
NPAD = 10240
RPS = NPAD // NS

R = 400
GRID = N // R


def _pre_kernel(xc_ref, xn_ref, wc_ref, bc_ref, wn_ref, bn_ref, wa_ref,
                ba_ref, nbrp_ref, acur_ref):
    wa = wa_ref[...]
    wa1 = wa[:H, :]
    wa2 = wa[H:, :]
    cur = jnp.dot(xc_ref[...], wc_ref[...],
                  preferred_element_type=jnp.float32) + bc_ref[...][None, :]
    nbr = jnp.dot(xn_ref[...], wn_ref[...],
                  preferred_element_type=jnp.float32) + bn_ref[...][None, :]
    acur = jnp.dot(cur, wa1, preferred_element_type=jnp.float32) + ba_ref[...]
    anbr = jnp.dot(nbr, wa2, preferred_element_type=jnp.float32)
    ones = jnp.ones((N, 1), jnp.float32)
    zeros = jnp.zeros((N, WPAD - H - 2), jnp.float32)
    nbrp_ref[...] = jnp.concatenate([nbr, ones, anbr, zeros], axis=1)
    acur_ref[...] = acur


def _pre(x_cur, x_nbr, W_cur, b_cur, W_nbr, b_nbr, W_att, b_att):
    return pl.pallas_call(
        _pre_kernel,
        out_shape=[
            jax.ShapeDtypeStruct((N, WPAD), jnp.float32),
            jax.ShapeDtypeStruct((N, 1), jnp.float32),
        ],
    )(x_cur, x_nbr, W_cur, b_cur, W_nbr, b_nbr, W_att, b_att)


def _sc_edge_kernel(acur_hbm, nbrp_hbm, src_hbm, dst_hbm, out_hbm,
                    acur_v, srcb_v, dstb_v, rows0, rows1, sbuf, wsum,
                    g0, g1, s0, s1):
    cid = lax.axis_index("c")
    sid = lax.axis_index("s")
    wid = cid * NS + sid
    stage = [pltpu.async_copy(acur_hbm, acur_v, g0),
             pltpu.async_copy(src_hbm.at[wid], srcb_v, g0),
             pltpu.async_copy(dst_hbm.at[wid], dstb_v, g0)]
    zero16 = jnp.zeros((L,), jnp.float32)

    def zrow(r, c):
        for cc in range(WPAD // L):
            rows0[r, pl.ds(cc * L, L)] = zero16
        return c

    lax.fori_loop(0, SB, zrow, 0)
    zc = [pltpu.async_copy(rows0.at[pl.ds(0, SB)],
                           wsum.at[pl.ds(sid * RPS + k * SB, SB)], g1)
          for k in range(RPS // SB)]
    for d in stage + zc:
        d.wait()
    plsc.subcore_barrier()

    def fire_gathers(b, buf, sem):
        for k in range(K):
            pltpu.async_copy(nbrp_hbm.at[dstb_v.at[b, k]],
                             buf.at[pl.ds(k * SB, SB)], sem)

    def drain_gathers(b, buf, sem):
        for k in range(K):
            pltpu.make_async_copy(nbrp_hbm.at[dstb_v.at[b, k]],
                                  buf.at[pl.ds(k * SB, SB)], sem).wait()

    def fire_scatters(b, buf, sem):
        for k in range(K):
            pltpu.async_copy(buf.at[pl.ds(k * SB, SB)],
                             wsum.at[srcb_v.at[b, k]], sem, add=True)

    def drain_scatters(b, buf, sem):
        for k in range(K):
            pltpu.make_async_copy(buf.at[pl.ds(k * SB, SB)],
                                  wsum.at[srcb_v.at[b, k]], sem).wait()

    iota16 = lax.iota(jnp.int32, L)
    c32 = jnp.full((L,), H, jnp.int32)
    c33 = jnp.full((L,), H + 1, jnp.int32)

    def compute(b, buf):
        def grp(g):
            evec = iota16 + g * L
            src16 = srcb_v[b, g // (SB // L), pl.ds((g % (SB // L)) * L, L)]
            sv = plsc.load_gather(acur_v, [src16])
            dv = plsc.load_gather(buf, [evec, c33])
            s = sv + dv
            s = jnp.where(s >= 0.0, s, 0.3 * s)
            wv = jnp.exp(s)
            plsc.store_scatter(sbuf, [evec, c32], wv)

            def colbody(j):
                cj = jnp.full((L,), 0, jnp.int32) + j
                col = plsc.load_gather(buf, [evec, cj])
                plsc.store_scatter(sbuf, [evec, cj], col * wv)

            plsc.parallel_loop(0, H, unroll=8)(colbody)

        plsc.parallel_loop(0, B // L, unroll=5)(grp)

    fire_gathers(0, rows0, g0)
    fire_gathers(1, rows1, g1)
    drain_gathers(0, rows0, g0)
    compute(0, rows0)
    fire_scatters(0, sbuf, s0)

    def pipe(p, c):
        b0 = 2 * p
        fire_gathers(b0 + 2, rows0, g0)
        drain_gathers(b0 + 1, rows1, g1)
        drain_scatters(b0, sbuf, s0)
        compute(b0 + 1, rows1)
        fire_scatters(b0 + 1, sbuf, s0)
        fire_gathers(b0 + 3, rows1, g1)
        drain_gathers(b0 + 2, rows0, g0)
        drain_scatters(b0 + 1, sbuf, s0)
        compute(b0 + 2, rows0)
        fire_scatters(b0 + 2, sbuf, s0)
        return c

    lax.fori_loop(0, (NB - 3) // 2, pipe, 0)
    fire_gathers(NB - 1, rows0, g0)
    drain_gathers(NB - 2, rows1, g1)
    drain_scatters(NB - 3, sbuf, s0)
    compute(NB - 2, rows1)
    fire_scatters(NB - 2, sbuf, s0)
    drain_gathers(NB - 1, rows0, g0)
    drain_scatters(NB - 2, sbuf, s0)
    compute(NB - 1, rows0)
    fire_scatters(NB - 1, sbuf, s0)
    drain_scatters(NB - 1, sbuf, s0)

    plsc.subcore_barrier()
    pltpu.sync_copy(wsum.at[pl.ds(sid * RPS, RPS)],
                    out_hbm.at[cid, pl.ds(sid * RPS, RPS)])


_sc_edges = functools.partial(
    pl.kernel,
    out_type=jax.ShapeDtypeStruct((NC, NPAD, WPAD), jnp.float32),
    mesh=plsc.VectorSubcoreMesh(core_axis_name="c", subcore_axis_name="s"),
    compiler_params=pltpu.CompilerParams(needs_layout_passes=False,
                                         use_tc_tiling_on_sc=False),
    scratch_types=[
        pltpu.VMEM((N,), jnp.float32),
        pltpu.VMEM((NB, K, SB), jnp.int32),
        pltpu.VMEM((NB, K, SB), jnp.int32),
        pltpu.VMEM((B, WPAD), jnp.float32),
        pltpu.VMEM((B, WPAD), jnp.float32),
        pltpu.VMEM((B, WPAD), jnp.float32),
        pltpu.VMEM_SHARED((NPAD, WPAD), jnp.float32),
        pltpu.SemaphoreType.DMA,
        pltpu.SemaphoreType.DMA,
        pltpu.SemaphoreType.DMA,
        pltpu.SemaphoreType.DMA,
    ],
)(_sc_edge_kernel)


def _post_kernel(p_ref, o_ref):
    p = p_ref[0, :N] + p_ref[1, :N]
    den = p[:, H:H + 1]
    num = p[:, 0:H]
    o_ref[...] = jnp.where(den > 0.0, num / den, 0.0)


def _post(parts):
    return pl.pallas_call(
        _post_kernel,
        out_shape=jax.ShapeDtypeStruct((N, H), jnp.float32),
    )(parts)


def kernel(x_cur, x_nbr, W_cur, b_cur, W_nbr, b_nbr, W_att, b_att, edges):
    nbrp, acur2d = _pre(x_cur, x_nbr, W_cur, b_cur, W_nbr, b_nbr, W_att, b_att)
    acur = acur2d.reshape(N)
    src = edges[:, 0].reshape(NW, NB, K, SB)
    dst = edges[:, 1].reshape(NW, NB, K, SB)
    parts = _sc_edges(acur, nbrp, src, dst)
    return _post(parts)
